# Initial kernel scaffold; baseline (speedup 1.0000x reference)
import jax
import jax.numpy as jnp
from jax.experimental import pallas as pl


def kernel(inputs, mask_position_ids, keep_position_ids, axis, mask_embedding):
    raise NotImplementedError("write your pallas kernel here")


# speedup vs baseline: 4.0761x; 4.0761x over previous
import functools

import jax
import jax.numpy as jnp
from jax import lax
from jax.experimental import pallas as pl
from jax.experimental.pallas import tpu as pltpu
from jax.experimental.pallas import tpu_sc as plsc

_NUM_CORES = 2
_NUM_SUBCORES = 16
_NW = _NUM_CORES * _NUM_SUBCORES


def _scatter_fill(x_flat, keep_glob, mask_glob, mask_rows, out_rows):
    nk, d = x_flat.shape
    nm = mask_glob.shape[0]
    mc = mask_rows.shape[0]
    assert nk % _NW == 0 and nm % _NW == 0
    nk_w = nk // _NW
    nm_w = nm // _NW
    c = 48
    while nk_w % c:
        c //= 2
    assert nk_w % c == 0 and nm_w % mc == 0 and c % 8 == 0 and mc % 8 == 0

    mesh = plsc.VectorSubcoreMesh(core_axis_name="c", subcore_axis_name="s")

    @functools.partial(
        pl.kernel,
        out_type=jax.ShapeDtypeStruct((out_rows, d), jnp.float32),
        mesh=mesh,
        scratch_types=[
            pltpu.VMEM((c, d), jnp.float32),
            pltpu.VMEM((c,), jnp.int32),
            pltpu.VMEM((mc, d), jnp.float32),
            pltpu.VMEM((mc,), jnp.int32),
        ],
    )
    def k(x_hbm, kidx_hbm, midx_hbm, mrows_hbm, out_hbm, dbuf, ibuf, mbuf, mibuf):
        wid = lax.axis_index("s") * _NUM_CORES + lax.axis_index("c")
        kbase = wid * nk_w
        mbase = wid * nm_w
        pltpu.sync_copy(mrows_hbm, mbuf)
        for j in range(nk_w // c):
            s = kbase + j * c
            pltpu.sync_copy(x_hbm.at[pl.ds(s, c)], dbuf)
            pltpu.sync_copy(kidx_hbm.at[pl.ds(s, c)], ibuf)
            pltpu.sync_copy(dbuf, out_hbm.at[ibuf])
        for t in range(nm_w // mc):
            pltpu.sync_copy(midx_hbm.at[pl.ds(mbase + t * mc, mc)], mibuf)
            pltpu.sync_copy(mbuf, out_hbm.at[mibuf])

    return k(x_flat, keep_glob, mask_glob, mask_rows)


def kernel(inputs, mask_position_ids, keep_position_ids, axis, mask_embedding):
    del axis
    inputs = inputs.astype(jnp.float32)
    b, lk, d = inputs.shape
    lm = mask_position_ids.shape[-1]
    length = lk + lm
    offs = (jnp.arange(b, dtype=jnp.int32) * length)[:, None]
    keep_glob = (keep_position_ids.astype(jnp.int32) + offs).reshape(b * lk)
    mask_glob = (mask_position_ids.astype(jnp.int32) + offs).reshape(b * lm)
    x_flat = inputs.reshape(b * lk, d)
    mc = 16
    mask_rows = jnp.broadcast_to(
        jnp.asarray(mask_embedding, dtype=jnp.float32)[None, :], (mc, d)
    )
    out_flat = _scatter_fill(x_flat, keep_glob, mask_glob, mask_rows, b * length)
    return out_flat.reshape(b, length, d)


# speedup vs baseline: 4.8749x; 1.1959x over previous
import functools

import jax
import jax.numpy as jnp
from jax import lax
from jax.experimental import pallas as pl
from jax.experimental.pallas import tpu as pltpu
from jax.experimental.pallas import tpu_sc as plsc

_NUM_CORES = 2
_NUM_SUBCORES = 16
_NW = _NUM_CORES * _NUM_SUBCORES


def _scatter_fill(x_flat, keep_glob, mask_glob, mask_rows, out_rows):
    nk, d = x_flat.shape
    nm = mask_glob.shape[0]
    mc = mask_rows.shape[0]
    assert nk % _NW == 0 and nm % _NW == 0
    nk_w = nk // _NW
    nm_w = nm // _NW
    c = 48
    while nk_w % c:
        c //= 2
    assert nk_w % c == 0 and nm_w % mc == 0 and c % 8 == 0 and mc % 8 == 0

    mesh = plsc.VectorSubcoreMesh(core_axis_name="c", subcore_axis_name="s")
    nch = nk_w // c
    nmch = nm_w // mc

    @functools.partial(
        pl.kernel,
        out_type=jax.ShapeDtypeStruct((out_rows, d), jnp.float32),
        mesh=mesh,
        scratch_types=[
            pltpu.VMEM((c, d), jnp.float32),
            pltpu.VMEM((c, d), jnp.float32),
            pltpu.VMEM((c,), jnp.int32),
            pltpu.VMEM((c,), jnp.int32),
            pltpu.VMEM((mc, d), jnp.float32),
            pltpu.VMEM((mc,), jnp.int32),
            pltpu.VMEM((mc,), jnp.int32),
        ]
        + [pltpu.SemaphoreType.DMA] * 11,
    )
    def k(x_hbm, kidx_hbm, midx_hbm, mrows_hbm, out_hbm,
          dbuf0, dbuf1, ibuf0, ibuf1, mbuf, mibuf0, mibuf1,
          sd0, sd1, si0, si1, ss0, ss1, smr, smi0, smi1, sms0, sms1):
        dbufs, ibufs = [dbuf0, dbuf1], [ibuf0, ibuf1]
        sd, si, ss = [sd0, sd1], [si0, si1], [ss0, ss1]
        mib, smi, sms = [mibuf0, mibuf1], [smi0, smi1], [sms0, sms1]
        wid = lax.axis_index("s") * _NUM_CORES + lax.axis_index("c")
        kbase = wid * nk_w
        mbase = wid * nm_w

        mstage = pltpu.async_copy(mrows_hbm, mbuf, smr)

        def start_mload(t):
            b = t & 1
            return pltpu.async_copy(
                midx_hbm.at[pl.ds(mbase + t * mc, mc)], mib[b], smi[b])

        mloads = {0: start_mload(0)}
        if nmch > 1:
            mloads[1] = start_mload(1)

        def start_load(j):
            b = j & 1
            s = kbase + j * c
            return (pltpu.async_copy(x_hbm.at[pl.ds(s, c)], dbufs[b], sd[b]),
                    pltpu.async_copy(kidx_hbm.at[pl.ds(s, c)], ibufs[b], si[b]))

        loads = {0: start_load(0)}
        if nch > 1:
            loads[1] = start_load(1)
        scats = {}
        for j in range(nch):
            b = j & 1
            loads[j][0].wait()
            loads[j][1].wait()
            scats[j] = pltpu.async_copy(dbufs[b], out_hbm.at[ibufs[b]], ss[b])
            if j + 2 < nch:
                scats[j].wait()
                loads[j + 2] = start_load(j + 2)
                del scats[j]
        mstage.wait()
        mscats = {}
        for t in range(nmch):
            b = t & 1
            mloads[t].wait()
            mscats[t] = pltpu.async_copy(mbuf, out_hbm.at[mib[b]], sms[b])
            if t + 2 < nmch:
                mscats[t].wait()
                mloads[t + 2] = start_mload(t + 2)
                del mscats[t]
        for h in list(scats.values()) + list(mscats.values()):
            h.wait()

    return k(x_flat, keep_glob, mask_glob, mask_rows)


def kernel(inputs, mask_position_ids, keep_position_ids, axis, mask_embedding):
    del axis
    inputs = inputs.astype(jnp.float32)
    b, lk, d = inputs.shape
    lm = mask_position_ids.shape[-1]
    length = lk + lm
    offs = (jnp.arange(b, dtype=jnp.int32) * length)[:, None]
    keep_glob = (keep_position_ids.astype(jnp.int32) + offs).reshape(b * lk)
    mask_glob = (mask_position_ids.astype(jnp.int32) + offs).reshape(b * lm)
    x_flat = inputs.reshape(b * lk, d)
    mc = 16
    mask_rows = jnp.broadcast_to(
        jnp.asarray(mask_embedding, dtype=jnp.float32)[None, :], (mc, d)
    )
    out_flat = _scatter_fill(x_flat, keep_glob, mask_glob, mask_rows, b * length)
    return out_flat.reshape(b, length, d)
